# baseline (speedup 1.0000x reference)
import jax
import jax.numpy as jnp
from jax.experimental import pallas as pl
from jax.experimental.pallas import tpu as pltpu

_MIN_NORM = 1e-15
_BALL_EPS = 4e-3
_HALF_LN2 = 0.34657359027997264

_INV_PIO2 = 0.6366197723675814
_DP1 = 1.5703125
_DP2 = 4.837512969970703125e-4
_DP3 = 7.549789948768648e-8


def _rowsum(x):
    return jnp.sum(x, axis=-1, keepdims=True)


def _fast_cos(x):
    qi = jnp.round(x * _INV_PIO2).astype(jnp.int32)
    qf = qi.astype(jnp.float32)
    r = x - qf * _DP1
    r = r - qf * _DP2
    r = r - qf * _DP3
    x2 = r * r
    cosp = ((2.443315711809948e-5 * x2 - 1.388731625493765e-3) * x2
            + 4.166664568298827e-2) * (x2 * x2) + (1.0 - 0.5 * x2)
    sinp = (((-1.9515295891e-4 * x2 + 8.3321608736e-3) * x2
             - 1.6666654611e-1) * x2) * r + r
    val = jnp.where((qi & 1) == 1, sinp, cosp)
    sign = ((qi + 1) & 2) << 30
    bits = pltpu.bitcast(val, jnp.int32) ^ sign
    return pltpu.bitcast(bits, jnp.float32)


def _norm_inv(x):
    ss = jnp.maximum(_rowsum(x * x), _MIN_NORM * _MIN_NORM)
    rn = jax.lax.rsqrt(ss)
    return ss, ss * rn, rn


def _aon(n, inv_n):
    nc = jnp.minimum(n, 1.0 - 1e-7)
    t = (1.0 + nc) / (1.0 - nc)
    return (_HALF_LN2 * inv_n) * jnp.log2(t)


def _gru_body(mi_ref, mem_ref, ts_ref, mts_ref, h_ref, tw_ref, tb_ref,
              wih_ref, whh_ref, bias_ref, nw_ref, nb_ref, out_ref):
    f32 = jnp.float32
    mi = mi_ref[...]
    hx = mem_ref[...]
    H = hx.shape[-1]

    dt = ts_ref[...] - mts_ref[...]
    ang = jnp.dot(dt, tw_ref[...], preferred_element_type=f32, precision=jax.lax.Precision.HIGHEST) + tb_ref[...]
    u = _fast_cos(ang)
    _, un, inv_un = _norm_inv(u)
    tn = jnp.tanh(un)
    maxnorm = 1.0 - _BALL_EPS
    pscale = jnp.where(tn > maxnorm, maxnorm / tn, 1.0)
    tf = (tn * inv_un * pscale) * u
    tfn = jnp.minimum(tn, maxnorm)

    mi2 = _rowsum(mi * mi)
    xss = jnp.maximum(mi2 + tfn * tfn, _MIN_NORM * _MIN_NORM)
    inv_xn = jax.lax.rsqrt(xss)
    xn = xss * inv_xn
    hss, hn, inv_hn = _norm_inv(hx)
    aox = _aon(xn, inv_xn)
    aoh = _aon(hn, inv_hn)

    x_cat = jnp.concatenate([mi, tf], axis=1)
    ux_all = jnp.dot(x_cat, wih_ref[...], preferred_element_type=f32, precision=jax.lax.Precision.HIGHEST)
    wh_all = jnp.dot(hx, whh_ref[...], preferred_element_type=f32, precision=jax.lax.Precision.HIGHEST)

    def mmv_post(m, aon_src):
        _, mxn, inv_mxn = _norm_inv(m)
        t = jnp.tanh(aon_src * mxn)
        return (t * inv_mxn) * m, t

    ux_r, t_uxr = mmv_post(ux_all[:, 0:H], aox)
    ux_h, t_uxh = mmv_post(ux_all[:, H:2 * H], aox)
    ux_z, t_uxz = mmv_post(ux_all[:, 2 * H:3 * H], aox)
    wh_r, t_whr = mmv_post(wh_all[:, 0:H], aoh)
    wh_z, t_whz = mmv_post(wh_all[:, 2 * H:3 * H], aoh)

    def madd(x, y, x2=None, y2=None):
        if x2 is None:
            x2 = _rowsum(x * x)
        if y2 is None:
            y2 = _rowsum(y * y)
        xy = _rowsum(x * y)
        num = (1.0 + 2.0 * xy + y2) * x + (1.0 - x2) * y
        inv_den = 1.0 / jnp.maximum(1.0 + 2.0 * xy + x2 * y2, _MIN_NORM)
        return num * inv_den

    b_r = bias_ref[0:1, :]
    b_h = bias_ref[1:2, :]
    b_z = bias_ref[2:3, :]
    b_r2 = _rowsum(b_r * b_r)
    b_h2 = _rowsum(b_h * b_h)
    b_z2 = _rowsum(b_z * b_z)

    gz = madd(madd(wh_z, ux_z, x2=t_whz * t_whz, y2=t_uxz * t_uxz), b_z, y2=b_z2)
    gr = madd(madd(wh_r, ux_r, x2=t_whr * t_whr, y2=t_uxr * t_uxr), b_r, y2=b_r2)

    def logmap_sig(y):
        _, n, inv_n = _norm_inv(y)
        return jax.nn.sigmoid(_aon(n, inv_n) * y)

    z = logmap_sig(gz)
    r = logmap_sig(gr)

    def mpm(w, x, aon_x):
        wx = w * x
        _, wxn, inv_wxn = _norm_inv(wx)
        t = jnp.tanh(aon_x * wxn)
        return (t * inv_wxn) * wx, t

    rh, t_rh = mpm(r, hx, aoh)
    rhn = jnp.maximum(t_rh, _MIN_NORM)
    aorh = _aon(rhn, 1.0 / rhn)
    wh_h = jnp.dot(rh, whh_ref[:, H:2 * H], preferred_element_type=f32, precision=jax.lax.Precision.HIGHEST)
    wh_h, t_whh = mmv_post(wh_h, aorh)
    h_tilde = madd(madd(wh_h, ux_h, x2=t_whh * t_whh, y2=t_uxh * t_uxh), b_h, y2=b_h2)

    delta = madd(-hx, h_tilde, x2=hss)
    _, dn, inv_dn = _norm_inv(delta)
    aod = _aon(dn, inv_dn)
    zd, t_zd = mpm(z, delta, aod)
    upd = madd(hx, zd, x2=hss, y2=t_zd * t_zd)

    hm = jnp.dot(h_ref[...], nw_ref[...], preferred_element_type=f32, precision=jax.lax.Precision.HIGHEST) + nb_ref[...]
    out_ref[...] = madd(upd, hm)


def kernel(mem_input, mem, ts, mem_ts, h, time_w, time_b,
           weight_ih, weight_hh, bias, node_W, node_b):
    N, D_in = mem_input.shape
    H = mem.shape[1]
    D_node = h.shape[1]
    D_t = time_w.shape[0]

    B = 512
    grid = (N // B,)

    wih_t = weight_ih.T
    whh_t = weight_hh.T
    nw_t = node_W.T
    ts2 = ts[:, None]
    mts2 = mem_ts[:, None]
    tw = time_w[None, :]
    tb = time_b[None, :]
    nb = node_b[None, :]

    fixed = lambda i: (0, 0)
    rows = lambda i: (i, 0)

    return pl.pallas_call(
        _gru_body,
        grid=grid,
        in_specs=[
            pl.BlockSpec((B, D_in), rows),
            pl.BlockSpec((B, H), rows),
            pl.BlockSpec((B, 1), rows),
            pl.BlockSpec((B, 1), rows),
            pl.BlockSpec((B, D_node), rows),
            pl.BlockSpec((1, D_t), fixed),
            pl.BlockSpec((1, D_t), fixed),
            pl.BlockSpec((D_in + D_t, 3 * H), fixed),
            pl.BlockSpec((H, 3 * H), fixed),
            pl.BlockSpec((3, H), fixed),
            pl.BlockSpec((D_node, H), fixed),
            pl.BlockSpec((1, H), fixed),
        ],
        out_specs=pl.BlockSpec((B, H), rows),
        out_shape=jax.ShapeDtypeStruct((N, H), jnp.float32),
        compiler_params=pltpu.CompilerParams(
            dimension_semantics=("parallel",),
        ),
    )(mem_input, mem, ts2, mts2, h, tw, tb, wih_t, whh_t, bias, nw_t, nb)


# speedup vs baseline: 1.7512x; 1.7512x over previous
import jax
import jax.numpy as jnp
from jax.experimental import pallas as pl
from jax.experimental.pallas import tpu as pltpu

_MIN_NORM = 1e-15
_BALL_EPS = 4e-3
_HALF_LN2 = 0.34657359027997264

_INV_PIO2 = 0.6366197723675814
_DP1 = 1.5703125
_DP2 = 4.837512969970703125e-4
_DP3 = 7.549789948768648e-8


def _cs(x):
    return jnp.sum(x, axis=0, keepdims=True)


def _fast_cos(x):
    qi = jnp.round(x * _INV_PIO2).astype(jnp.int32)
    qf = qi.astype(jnp.float32)
    r = x - qf * _DP1
    r = r - qf * _DP2
    r = r - qf * _DP3
    x2 = r * r
    cosp = ((2.443315711809948e-5 * x2 - 1.388731625493765e-3) * x2
            + 4.166664568298827e-2) * (x2 * x2) + (1.0 - 0.5 * x2)
    sinp = (((-1.9515295891e-4 * x2 + 8.3321608736e-3) * x2
             - 1.6666654611e-1) * x2) * r + r
    val = jnp.where((qi & 1) == 1, sinp, cosp)
    sign = ((qi + 1) & 2) << 30
    bits = pltpu.bitcast(val, jnp.int32) ^ sign
    return pltpu.bitcast(bits, jnp.float32)


def _norm_inv(x):
    ss = jnp.maximum(_cs(x * x), _MIN_NORM * _MIN_NORM)
    rn = jax.lax.rsqrt(ss)
    return ss, ss * rn, rn


def _aon(n, inv_n):
    nc = jnp.minimum(n, 1.0 - 1e-7)
    t = (1.0 + nc) / (1.0 - nc)
    return (_HALF_LN2 * inv_n) * jnp.log2(t)


def _gru_body(mi_ref, mem_ref, ts_ref, mts_ref, h_ref, twb_ref,
              wih_ref, whh_ref, biasT_ref, nwb_ref, out_ref):
    f32 = jnp.float32
    hp = jax.lax.Precision.HIGHEST
    mi = mi_ref[...]
    hx = mem_ref[...]
    H = hx.shape[0]

    dt = ts_ref[...] - mts_ref[...]
    dt1 = jnp.concatenate([dt, jnp.ones_like(dt)], axis=0)
    ang = jnp.dot(twb_ref[...], dt1, preferred_element_type=f32, precision=hp)
    u = _fast_cos(ang)
    _, un, inv_un = _norm_inv(u)
    tn = jnp.tanh(un)
    maxnorm = 1.0 - _BALL_EPS
    pscale = jnp.where(tn > maxnorm, maxnorm / tn, 1.0)
    tf = (tn * inv_un * pscale) * u
    tfn = jnp.minimum(tn, maxnorm)

    mi2 = _cs(mi * mi)
    xss = jnp.maximum(mi2 + tfn * tfn, _MIN_NORM * _MIN_NORM)
    inv_xn = jax.lax.rsqrt(xss)
    xn = xss * inv_xn
    hss, hn, inv_hn = _norm_inv(hx)
    aox = _aon(xn, inv_xn)
    aoh = _aon(hn, inv_hn)

    xT = jnp.concatenate([mi, tf], axis=0)
    ux_all = jnp.dot(wih_ref[...], xT, preferred_element_type=f32)
    wh_all = jnp.dot(whh_ref[...], hx, preferred_element_type=f32)

    def mmv_post(m, aon_src):
        _, mxn, inv_mxn = _norm_inv(m)
        t = jnp.tanh(aon_src * mxn)
        return (t * inv_mxn) * m, t

    ux_r, t_uxr = mmv_post(ux_all[0:H], aox)
    ux_h, t_uxh = mmv_post(ux_all[H:2 * H], aox)
    ux_z, t_uxz = mmv_post(ux_all[2 * H:3 * H], aox)
    wh_r, t_whr = mmv_post(wh_all[0:H], aoh)
    wh_z, t_whz = mmv_post(wh_all[2 * H:3 * H], aoh)

    def madd(x, y, x2=None, y2=None):
        if x2 is None:
            x2 = _cs(x * x)
        if y2 is None:
            y2 = _cs(y * y)
        xy = _cs(x * y)
        num = (1.0 + 2.0 * xy + y2) * x + (1.0 - x2) * y
        inv_den = 1.0 / jnp.maximum(1.0 + 2.0 * xy + x2 * y2, _MIN_NORM)
        return num * inv_den

    ones_row = jnp.ones_like(dt)
    biasT = biasT_ref[...]
    b2_all = _cs(biasT * biasT)

    def b_bcast(k):
        col = biasT[:, k:k + 1]
        return (jnp.dot(col, ones_row, preferred_element_type=f32, precision=hp),
                b2_all[:, k:k + 1])

    bb_r, b_r2 = b_bcast(0)
    bb_h, b_h2 = b_bcast(1)
    bb_z, b_z2 = b_bcast(2)

    gz = madd(madd(wh_z, ux_z, x2=t_whz * t_whz, y2=t_uxz * t_uxz), bb_z, y2=b_z2)
    gr = madd(madd(wh_r, ux_r, x2=t_whr * t_whr, y2=t_uxr * t_uxr), bb_r, y2=b_r2)

    def logmap_sig(y):
        _, n, inv_n = _norm_inv(y)
        return jax.nn.sigmoid(_aon(n, inv_n) * y)

    z = logmap_sig(gz)
    r = logmap_sig(gr)

    def mpm(w, x, aon_x):
        wx = w * x
        _, wxn, inv_wxn = _norm_inv(wx)
        t = jnp.tanh(aon_x * wxn)
        return (t * inv_wxn) * wx, t

    rh, t_rh = mpm(r, hx, aoh)
    rhn = jnp.maximum(t_rh, _MIN_NORM)
    aorh = _aon(rhn, 1.0 / rhn)
    wh_h = jnp.dot(whh_ref[H:2 * H, :], rh, preferred_element_type=f32)
    wh_h, t_whh = mmv_post(wh_h, aorh)
    h_tilde = madd(madd(wh_h, ux_h, x2=t_whh * t_whh, y2=t_uxh * t_uxh), bb_h, y2=b_h2)

    delta = madd(-hx, h_tilde, x2=hss)
    _, dn, inv_dn = _norm_inv(delta)
    aod = _aon(dn, inv_dn)
    zd, t_zd = mpm(z, delta, aod)
    upd = madd(hx, zd, x2=hss, y2=t_zd * t_zd)

    hm = jnp.dot(nwb_ref[...], h_ref[...], preferred_element_type=f32)
    out_ref[...] = madd(upd, hm)


def kernel(mem_input, mem, ts, mem_ts, h, time_w, time_b,
           weight_ih, weight_hh, bias, node_W, node_b):
    N, D_in = mem_input.shape
    H = mem.shape[1]
    D_node = h.shape[1]
    D_t = time_w.shape[0]

    B = 512
    grid = (N // B,)

    miT = mem_input.T
    memT = mem.T
    hT1 = jnp.concatenate([h.T, jnp.ones((1, N), jnp.float32)], axis=0)
    ts_r = ts[None, :]
    mts_r = mem_ts[None, :]
    twb = jnp.stack([time_w, time_b], axis=1)
    biasT = bias.T
    nwb = jnp.concatenate([node_W, node_b[:, None]], axis=1)

    fixed = lambda i: (0, 0)
    cols = lambda i: (0, i)

    outT = pl.pallas_call(
        _gru_body,
        grid=grid,
        in_specs=[
            pl.BlockSpec((D_in, B), cols),
            pl.BlockSpec((H, B), cols),
            pl.BlockSpec((1, B), cols),
            pl.BlockSpec((1, B), cols),
            pl.BlockSpec((D_node + 1, B), cols),
            pl.BlockSpec((D_t, 2), fixed),
            pl.BlockSpec((3 * H, D_in + D_t), fixed),
            pl.BlockSpec((3 * H, H), fixed),
            pl.BlockSpec((H, 3), fixed),
            pl.BlockSpec((H, D_node + 1), fixed),
        ],
        out_specs=pl.BlockSpec((H, B), cols),
        out_shape=jax.ShapeDtypeStruct((H, N), jnp.float32),
        compiler_params=pltpu.CompilerParams(
            dimension_semantics=("parallel",),
        ),
    )(miT, memT, ts_r, mts_r, hT1, twb, weight_ih, weight_hh, biasT, nwb)
    return outT.T


# speedup vs baseline: 2.0121x; 1.1490x over previous
import jax
import jax.numpy as jnp
from jax.experimental import pallas as pl
from jax.experimental.pallas import tpu as pltpu

_MIN_NORM = 1e-15
_BALL_EPS = 4e-3
_HALF_LN2 = 0.34657359027997264

_INV_PIO2 = 0.6366197723675814
_DP1 = 1.5703125
_DP2 = 4.837512969970703125e-4
_DP3 = 7.549789948768648e-8


def _cs(x):
    return jnp.sum(x, axis=0, keepdims=True)


def _fast_cos(x):
    qi = jnp.round(x * _INV_PIO2).astype(jnp.int32)
    qf = qi.astype(jnp.float32)
    r = x - qf * _DP1
    r = r - qf * _DP2
    r = r - qf * _DP3
    x2 = r * r
    cosp = ((2.443315711809948e-5 * x2 - 1.388731625493765e-3) * x2
            + 4.166664568298827e-2) * (x2 * x2) + (1.0 - 0.5 * x2)
    sinp = (((-1.9515295891e-4 * x2 + 8.3321608736e-3) * x2
             - 1.6666654611e-1) * x2) * r + r
    val = jnp.where((qi & 1) == 1, sinp, cosp)
    sign = ((qi + 1) & 2) << 30
    bits = pltpu.bitcast(val, jnp.int32) ^ sign
    return pltpu.bitcast(bits, jnp.float32)


def _norm_inv(x):
    ss = jnp.maximum(_cs(x * x), _MIN_NORM * _MIN_NORM)
    rn = jax.lax.rsqrt(ss)
    return ss, ss * rn, rn


def _aon(n, inv_n):
    nc = jnp.minimum(n, 1.0 - 1e-7)
    t = (1.0 + nc) / (1.0 - nc)
    return (_HALF_LN2 * inv_n) * jnp.log2(t)


def _gru_body(mi_ref, mem_ref, ts_ref, mts_ref, h_ref, twb_ref,
              wih_ref, whh_ref, biasT_ref, nw_ref, nbc_ref, out_ref):
    f32 = jnp.float32
    hp = jax.lax.Precision.HIGHEST
    mi = mi_ref[...].T
    hx = mem_ref[...].T
    H = hx.shape[0]

    dt = ts_ref[...] - mts_ref[...]
    dt1 = jnp.concatenate([dt, jnp.ones_like(dt)], axis=0)
    ang = jnp.dot(twb_ref[...], dt1, preferred_element_type=f32, precision=hp)
    u = _fast_cos(ang)
    _, un, inv_un = _norm_inv(u)
    tn = jnp.tanh(un)
    maxnorm = 1.0 - _BALL_EPS
    pscale = jnp.where(tn > maxnorm, maxnorm / tn, 1.0)
    tf = (tn * inv_un * pscale) * u
    tfn = jnp.minimum(tn, maxnorm)

    mi2 = _cs(mi * mi)
    xss = jnp.maximum(mi2 + tfn * tfn, _MIN_NORM * _MIN_NORM)
    inv_xn = jax.lax.rsqrt(xss)
    xn = xss * inv_xn
    hss, hn, inv_hn = _norm_inv(hx)
    aox = _aon(xn, inv_xn)
    aoh = _aon(hn, inv_hn)

    xT = jnp.concatenate([mi, tf], axis=0)
    ux_all = jnp.dot(wih_ref[...], xT, preferred_element_type=f32)
    wh_all = jnp.dot(whh_ref[...], hx, preferred_element_type=f32)

    def mmv_post(m, aon_src):
        _, mxn, inv_mxn = _norm_inv(m)
        t = jnp.tanh(aon_src * mxn)
        return (t * inv_mxn) * m, t

    ux_r, t_uxr = mmv_post(ux_all[0:H], aox)
    ux_h, t_uxh = mmv_post(ux_all[H:2 * H], aox)
    ux_z, t_uxz = mmv_post(ux_all[2 * H:3 * H], aox)
    wh_r, t_whr = mmv_post(wh_all[0:H], aoh)
    wh_z, t_whz = mmv_post(wh_all[2 * H:3 * H], aoh)

    def madd(x, y, x2=None, y2=None):
        if x2 is None:
            x2 = _cs(x * x)
        if y2 is None:
            y2 = _cs(y * y)
        xy = _cs(x * y)
        num = (1.0 + 2.0 * xy + y2) * x + (1.0 - x2) * y
        inv_den = 1.0 / jnp.maximum(1.0 + 2.0 * xy + x2 * y2, _MIN_NORM)
        return num * inv_den

    ones_row = jnp.ones_like(dt)
    biasT = biasT_ref[...]
    b2_all = _cs(biasT * biasT)

    def b_bcast(k):
        col = biasT[:, k:k + 1]
        return (jnp.dot(col, ones_row, preferred_element_type=f32, precision=hp),
                b2_all[:, k:k + 1])

    bb_r, b_r2 = b_bcast(0)
    bb_h, b_h2 = b_bcast(1)
    bb_z, b_z2 = b_bcast(2)

    gz = madd(madd(wh_z, ux_z, x2=t_whz * t_whz, y2=t_uxz * t_uxz), bb_z, y2=b_z2)
    gr = madd(madd(wh_r, ux_r, x2=t_whr * t_whr, y2=t_uxr * t_uxr), bb_r, y2=b_r2)

    def logmap_sig(y):
        _, n, inv_n = _norm_inv(y)
        return jax.nn.sigmoid(_aon(n, inv_n) * y)

    z = logmap_sig(gz)
    r = logmap_sig(gr)

    def mpm(w, x, aon_x):
        wx = w * x
        _, wxn, inv_wxn = _norm_inv(wx)
        t = jnp.tanh(aon_x * wxn)
        return (t * inv_wxn) * wx, t

    rh, t_rh = mpm(r, hx, aoh)
    rhn = jnp.maximum(t_rh, _MIN_NORM)
    aorh = _aon(rhn, 1.0 / rhn)
    wh_h = jnp.dot(whh_ref[H:2 * H, :], rh, preferred_element_type=f32)
    wh_h, t_whh = mmv_post(wh_h, aorh)
    h_tilde = madd(madd(wh_h, ux_h, x2=t_whh * t_whh, y2=t_uxh * t_uxh), bb_h, y2=b_h2)

    delta = madd(-hx, h_tilde, x2=hss)
    _, dn, inv_dn = _norm_inv(delta)
    aod = _aon(dn, inv_dn)
    zd, t_zd = mpm(z, delta, aod)
    upd = madd(hx, zd, x2=hss, y2=t_zd * t_zd)

    nb_bcast = jnp.dot(nbc_ref[...], ones_row, preferred_element_type=f32,
                       precision=hp)
    hm = jnp.dot(nw_ref[...], h_ref[...].T, preferred_element_type=f32) + nb_bcast
    out_ref[...] = madd(upd, hm).T


def kernel(mem_input, mem, ts, mem_ts, h, time_w, time_b,
           weight_ih, weight_hh, bias, node_W, node_b):
    N, D_in = mem_input.shape
    H = mem.shape[1]
    D_node = h.shape[1]
    D_t = time_w.shape[0]

    B = 512
    grid = (N // B,)

    ts_r = ts[None, :]
    mts_r = mem_ts[None, :]
    twb = jnp.stack([time_w, time_b], axis=1)
    biasT = bias.T
    nbc = node_b[:, None]

    fixed = lambda i: (0, 0)
    cols = lambda i: (0, i)
    rows = lambda i: (i, 0)

    return pl.pallas_call(
        _gru_body,
        grid=grid,
        in_specs=[
            pl.BlockSpec((B, D_in), rows),
            pl.BlockSpec((B, H), rows),
            pl.BlockSpec((1, B), cols),
            pl.BlockSpec((1, B), cols),
            pl.BlockSpec((B, D_node), rows),
            pl.BlockSpec((D_t, 2), fixed),
            pl.BlockSpec((3 * H, D_in + D_t), fixed),
            pl.BlockSpec((3 * H, H), fixed),
            pl.BlockSpec((H, 3), fixed),
            pl.BlockSpec((H, D_node), fixed),
            pl.BlockSpec((H, 1), fixed),
        ],
        out_specs=pl.BlockSpec((B, H), rows),
        out_shape=jax.ShapeDtypeStruct((N, H), jnp.float32),
        compiler_params=pltpu.CompilerParams(
            dimension_semantics=("parallel",),
        ),
    )(mem_input, mem, ts_r, mts_r, h, twb, weight_ih, weight_hh, biasT,
      node_W, nbc)


# speedup vs baseline: 2.3282x; 1.1571x over previous
import jax
import jax.numpy as jnp
from jax.experimental import pallas as pl
from jax.experimental.pallas import tpu as pltpu

_MIN_NORM = 1e-15
_BALL_EPS = 4e-3
_HALF_LN2 = 0.34657359027997264

_INV_PI = 0.3183098861837907
_PI1 = 3.140625
_PI2 = 9.675025939941406e-4
_PI3 = 1.509957990978376e-7


def _cs(x):
    return jnp.sum(x, axis=0, keepdims=True)


def _fast_cos(x):
    qi = jnp.round(x * _INV_PI).astype(jnp.int32)
    qf = qi.astype(jnp.float32)
    r = x - qf * _PI1
    r = r - qf * _PI2
    r = r - qf * _PI3
    z = r * r
    p = 2.08767569878681e-9
    p = p * z - 2.7557319223985893e-7
    p = p * z + 2.48015873015873e-5
    p = p * z - 1.3888888888888887e-3
    p = p * z + 4.1666666666666664e-2
    val = (p * z - 0.5) * z + 1.0
    sign = (qi & 1) << 31
    bits = pltpu.bitcast(val, jnp.int32) ^ sign
    return pltpu.bitcast(bits, jnp.float32)


def _norm_inv(x):
    ss = jnp.maximum(_cs(x * x), _MIN_NORM * _MIN_NORM)
    rn = jax.lax.rsqrt(ss)
    return ss, ss * rn, rn


def _aon(n, inv_n):
    nc = jnp.minimum(n, 1.0 - 1e-7)
    t = (1.0 + nc) / (1.0 - nc)
    return (_HALF_LN2 * inv_n) * jnp.log2(t)


def _gru_body(mi_ref, mem_ref, ts_ref, mts_ref, h_ref, twb_ref,
              wih_ref, whh_ref, biasT_ref, nw_ref, nbc_ref, out_ref):
    f32 = jnp.float32
    hp = jax.lax.Precision.HIGHEST
    mi = mi_ref[...].T
    hx = mem_ref[...].T
    H = hx.shape[0]

    dt = ts_ref[...] - mts_ref[...]
    dt1 = jnp.concatenate([dt, jnp.ones_like(dt)], axis=0)
    ang = jnp.dot(twb_ref[...], dt1, preferred_element_type=f32, precision=hp)
    u = _fast_cos(ang)
    _, un, inv_un = _norm_inv(u)
    tn = jnp.tanh(un)
    maxnorm = 1.0 - _BALL_EPS
    pscale = jnp.where(tn > maxnorm, maxnorm / tn, 1.0)
    tf = (tn * inv_un * pscale) * u
    tfn = jnp.minimum(tn, maxnorm)

    mi2 = _cs(mi * mi)
    xss = jnp.maximum(mi2 + tfn * tfn, _MIN_NORM * _MIN_NORM)
    inv_xn = jax.lax.rsqrt(xss)
    xn = xss * inv_xn
    hss, hn, inv_hn = _norm_inv(hx)
    aox = _aon(xn, inv_xn)
    aoh = _aon(hn, inv_hn)

    xT = jnp.concatenate([mi, tf], axis=0)
    ux_all = jnp.dot(wih_ref[...], xT, preferred_element_type=f32)
    wh_all = jnp.dot(whh_ref[...], hx, preferred_element_type=f32)

    def mmv_post(m, aon_src):
        _, mxn, inv_mxn = _norm_inv(m)
        t = jnp.tanh(aon_src * mxn)
        return (t * inv_mxn) * m, t

    ux_r, t_uxr = mmv_post(ux_all[0:H], aox)
    ux_h, t_uxh = mmv_post(ux_all[H:2 * H], aox)
    ux_z, t_uxz = mmv_post(ux_all[2 * H:3 * H], aox)
    wh_r, t_whr = mmv_post(wh_all[0:H], aoh)
    wh_z, t_whz = mmv_post(wh_all[2 * H:3 * H], aoh)

    def madd(x, y, x2=None, y2=None):
        if x2 is None:
            x2 = _cs(x * x)
        if y2 is None:
            y2 = _cs(y * y)
        xy = _cs(x * y)
        num = (1.0 + 2.0 * xy + y2) * x + (1.0 - x2) * y
        inv_den = 1.0 / jnp.maximum(1.0 + 2.0 * xy + x2 * y2, _MIN_NORM)
        return num * inv_den

    ones_row = jnp.ones_like(dt)
    biasT = biasT_ref[...]
    b2_all = _cs(biasT * biasT)

    def b_bcast(k):
        col = biasT[:, k:k + 1]
        return (jnp.dot(col, ones_row, preferred_element_type=f32),
                b2_all[:, k:k + 1])

    bb_r, b_r2 = b_bcast(0)
    bb_h, b_h2 = b_bcast(1)
    bb_z, b_z2 = b_bcast(2)

    gz = madd(madd(wh_z, ux_z, x2=t_whz * t_whz, y2=t_uxz * t_uxz), bb_z, y2=b_z2)
    gr = madd(madd(wh_r, ux_r, x2=t_whr * t_whr, y2=t_uxr * t_uxr), bb_r, y2=b_r2)

    def logmap_sig(y):
        _, n, inv_n = _norm_inv(y)
        return jax.nn.sigmoid(_aon(n, inv_n) * y)

    z = logmap_sig(gz)
    r = logmap_sig(gr)

    def mpm(w, x, aon_x):
        wx = w * x
        _, wxn, inv_wxn = _norm_inv(wx)
        t = jnp.tanh(aon_x * wxn)
        return (t * inv_wxn) * wx, t

    rh, t_rh = mpm(r, hx, aoh)
    rhn = jnp.maximum(t_rh, _MIN_NORM)
    aorh = _aon(rhn, 1.0 / rhn)
    wh_h = jnp.dot(whh_ref[H:2 * H, :], rh, preferred_element_type=f32)
    wh_h, t_whh = mmv_post(wh_h, aorh)
    h_tilde = madd(madd(wh_h, ux_h, x2=t_whh * t_whh, y2=t_uxh * t_uxh), bb_h, y2=b_h2)

    delta = madd(-hx, h_tilde, x2=hss)
    _, dn, inv_dn = _norm_inv(delta)
    aod = _aon(dn, inv_dn)
    zd, t_zd = mpm(z, delta, aod)
    upd = madd(hx, zd, x2=hss, y2=t_zd * t_zd)

    nb_bcast = jnp.dot(nbc_ref[...], ones_row, preferred_element_type=f32)
    hm = jnp.dot(nw_ref[...], h_ref[...].T, preferred_element_type=f32) + nb_bcast
    out_ref[...] = madd(upd, hm).T


def kernel(mem_input, mem, ts, mem_ts, h, time_w, time_b,
           weight_ih, weight_hh, bias, node_W, node_b):
    N, D_in = mem_input.shape
    H = mem.shape[1]
    D_node = h.shape[1]
    D_t = time_w.shape[0]

    B = 1024
    grid = (N // B,)

    ts_r = ts[None, :]
    mts_r = mem_ts[None, :]
    twb = jnp.stack([time_w, time_b], axis=1)
    biasT = bias.T
    nbc = node_b[:, None]

    fixed = lambda i: (0, 0)
    cols = lambda i: (0, i)
    rows = lambda i: (i, 0)

    return pl.pallas_call(
        _gru_body,
        grid=grid,
        in_specs=[
            pl.BlockSpec((B, D_in), rows),
            pl.BlockSpec((B, H), rows),
            pl.BlockSpec((1, B), cols),
            pl.BlockSpec((1, B), cols),
            pl.BlockSpec((B, D_node), rows),
            pl.BlockSpec((D_t, 2), fixed),
            pl.BlockSpec((3 * H, D_in + D_t), fixed),
            pl.BlockSpec((3 * H, H), fixed),
            pl.BlockSpec((H, 3), fixed),
            pl.BlockSpec((H, D_node), fixed),
            pl.BlockSpec((H, 1), fixed),
        ],
        out_specs=pl.BlockSpec((B, H), rows),
        out_shape=jax.ShapeDtypeStruct((N, H), jnp.float32),
        compiler_params=pltpu.CompilerParams(
            dimension_semantics=("parallel",),
        ),
    )(mem_input, mem, ts_r, mts_r, h, twb, weight_ih, weight_hh, biasT,
      node_W, nbc)


# speedup vs baseline: 2.4952x; 1.0717x over previous
import jax
import jax.numpy as jnp
from jax.experimental import pallas as pl
from jax.experimental.pallas import tpu as pltpu

_MIN_NORM = 1e-15
_BALL_EPS = 4e-3
_HALF_LN2 = 0.34657359027997264

_INV_PI = 0.3183098861837907
_PI1 = 3.140625
_PI2 = 9.675025939941406e-4
_PI3 = 1.509957990978376e-7


def _cs(x):
    return jnp.sum(x, axis=0, keepdims=True)


def _fast_cos(x):
    qi = jnp.round(x * _INV_PI).astype(jnp.int32)
    qf = qi.astype(jnp.float32)
    r = x - qf * _PI1
    r = r - qf * _PI2
    r = r - qf * _PI3
    z = r * r
    p = 2.08767569878681e-9
    p = p * z - 2.7557319223985893e-7
    p = p * z + 2.48015873015873e-5
    p = p * z - 1.3888888888888887e-3
    p = p * z + 4.1666666666666664e-2
    val = (p * z - 0.5) * z + 1.0
    sign = (qi & 1) << 31
    bits = pltpu.bitcast(val, jnp.int32) ^ sign
    return pltpu.bitcast(bits, jnp.float32)


def _norm_inv(x):
    ss = jnp.maximum(_cs(x * x), _MIN_NORM * _MIN_NORM)
    rn = jax.lax.rsqrt(ss)
    return ss, ss * rn, rn


def _aon(n, inv_n):
    nc = jnp.minimum(n, 1.0 - 1e-7)
    t = (1.0 + nc) / (1.0 - nc)
    return (_HALF_LN2 * inv_n) * jnp.log2(t)


def _gru_body(mi_ref, mem_ref, ts_ref, mts_ref, h_ref, twb_ref, tbb_ref,
              wih_ref, whh_ref, biasT_ref, bbr_ref, bbh_ref, bbz_ref,
              nw_ref, nbb_ref, out_ref):
    f32 = jnp.float32
    hp = jax.lax.Precision.HIGHEST
    mi = mi_ref[...].T
    hx = mem_ref[...].T
    H = hx.shape[0]

    dt = ts_ref[...] - mts_ref[...]
    ang = dt * twb_ref[...] + tbb_ref[...]
    u = _fast_cos(ang)
    _, un, inv_un = _norm_inv(u)
    tn = jnp.tanh(un)
    maxnorm = 1.0 - _BALL_EPS
    pscale = jnp.where(tn > maxnorm, maxnorm / tn, 1.0)
    tf = (tn * inv_un * pscale) * u
    tfn = jnp.minimum(tn, maxnorm)

    mi2 = _cs(mi * mi)
    xss = jnp.maximum(mi2 + tfn * tfn, _MIN_NORM * _MIN_NORM)
    inv_xn = jax.lax.rsqrt(xss)
    xn = xss * inv_xn
    hss, hn, inv_hn = _norm_inv(hx)
    aox = _aon(xn, inv_xn)
    aoh = _aon(hn, inv_hn)

    xT = jnp.concatenate([mi, tf], axis=0)
    ux_all = jnp.dot(wih_ref[...], xT, preferred_element_type=f32)
    wh_all = jnp.dot(whh_ref[...], hx, preferred_element_type=f32)

    def mmv_post(m, aon_src):
        _, mxn, inv_mxn = _norm_inv(m)
        t = jnp.tanh(aon_src * mxn)
        return (t * inv_mxn) * m, t

    ux_r, t_uxr = mmv_post(ux_all[0:H], aox)
    ux_h, t_uxh = mmv_post(ux_all[H:2 * H], aox)
    ux_z, t_uxz = mmv_post(ux_all[2 * H:3 * H], aox)
    wh_r, t_whr = mmv_post(wh_all[0:H], aoh)
    wh_z, t_whz = mmv_post(wh_all[2 * H:3 * H], aoh)

    def madd(x, y, x2=None, y2=None):
        if x2 is None:
            x2 = _cs(x * x)
        if y2 is None:
            y2 = _cs(y * y)
        xy = _cs(x * y)
        num = (1.0 + 2.0 * xy + y2) * x + (1.0 - x2) * y
        inv_den = 1.0 / jnp.maximum(1.0 + 2.0 * xy + x2 * y2, _MIN_NORM)
        return num * inv_den

    biasT = biasT_ref[...]
    b2_all = _cs(biasT * biasT)
    bb_r, b_r2 = bbr_ref[...], b2_all[:, 0:1]
    bb_h, b_h2 = bbh_ref[...], b2_all[:, 1:2]
    bb_z, b_z2 = bbz_ref[...], b2_all[:, 2:3]

    gz = madd(madd(wh_z, ux_z, x2=t_whz * t_whz, y2=t_uxz * t_uxz), bb_z, y2=b_z2)
    gr = madd(madd(wh_r, ux_r, x2=t_whr * t_whr, y2=t_uxr * t_uxr), bb_r, y2=b_r2)

    def logmap_sig(y):
        _, n, inv_n = _norm_inv(y)
        return jax.nn.sigmoid(_aon(n, inv_n) * y)

    z = logmap_sig(gz)
    r = logmap_sig(gr)

    def mpm(w, x, aon_x):
        wx = w * x
        _, wxn, inv_wxn = _norm_inv(wx)
        t = jnp.tanh(aon_x * wxn)
        return (t * inv_wxn) * wx, t

    rh, t_rh = mpm(r, hx, aoh)
    rhn = jnp.maximum(t_rh, _MIN_NORM)
    aorh = _aon(rhn, 1.0 / rhn)
    wh_h = jnp.dot(whh_ref[H:2 * H, :], rh, preferred_element_type=f32)
    wh_h, t_whh = mmv_post(wh_h, aorh)
    h_tilde = madd(madd(wh_h, ux_h, x2=t_whh * t_whh, y2=t_uxh * t_uxh), bb_h, y2=b_h2)

    delta = madd(-hx, h_tilde, x2=hss)
    _, dn, inv_dn = _norm_inv(delta)
    aod = _aon(dn, inv_dn)
    zd, t_zd = mpm(z, delta, aod)
    upd = madd(hx, zd, x2=hss, y2=t_zd * t_zd)

    hm = jnp.dot(nw_ref[...], h_ref[...].T, preferred_element_type=f32) + nbb_ref[...]
    out_ref[...] = madd(upd, hm).T


def kernel(mem_input, mem, ts, mem_ts, h, time_w, time_b,
           weight_ih, weight_hh, bias, node_W, node_b):
    N, D_in = mem_input.shape
    H = mem.shape[1]
    D_node = h.shape[1]
    D_t = time_w.shape[0]

    B = 1024
    grid = (N // B,)

    ts_r = ts[None, :]
    mts_r = mem_ts[None, :]
    twb = jnp.broadcast_to(time_w[:, None], (D_t, B))
    tbb = jnp.broadcast_to(time_b[:, None], (D_t, B))
    biasT = bias.T
    bbr = jnp.broadcast_to(bias[0][:, None], (H, B))
    bbh = jnp.broadcast_to(bias[1][:, None], (H, B))
    bbz = jnp.broadcast_to(bias[2][:, None], (H, B))
    nbb = jnp.broadcast_to(node_b[:, None], (H, B))

    fixed = lambda i: (0, 0)
    cols = lambda i: (0, i)
    rows = lambda i: (i, 0)

    return pl.pallas_call(
        _gru_body,
        grid=grid,
        in_specs=[
            pl.BlockSpec((B, D_in), rows),
            pl.BlockSpec((B, H), rows),
            pl.BlockSpec((1, B), cols),
            pl.BlockSpec((1, B), cols),
            pl.BlockSpec((B, D_node), rows),
            pl.BlockSpec((D_t, B), fixed),
            pl.BlockSpec((D_t, B), fixed),
            pl.BlockSpec((3 * H, D_in + D_t), fixed),
            pl.BlockSpec((3 * H, H), fixed),
            pl.BlockSpec((H, 3), fixed),
            pl.BlockSpec((H, B), fixed),
            pl.BlockSpec((H, B), fixed),
            pl.BlockSpec((H, B), fixed),
            pl.BlockSpec((H, D_node), fixed),
            pl.BlockSpec((H, B), fixed),
        ],
        out_specs=pl.BlockSpec((B, H), rows),
        out_shape=jax.ShapeDtypeStruct((N, H), jnp.float32),
        compiler_params=pltpu.CompilerParams(
            dimension_semantics=("parallel",),
        ),
    )(mem_input, mem, ts_r, mts_r, h, twb, tbb, weight_ih, weight_hh, biasT,
      bbr, bbh, bbz, node_W, nbb)
